# baseline (speedup 1.0000x reference)
import functools

import jax
import jax.numpy as jnp
from jax import lax
from jax.experimental import pallas as pl
from jax.experimental.pallas import tpu as pltpu
from jax.experimental.pallas import tpu_sc as plsc

N = 10000
E = 320000
D = 128
HID = 128
HC = 64
NCLS = 8

NW = 32
EPW = E // NW
C = 40
NCH = EPW // C
TPS = 16
STRIPE = 624
ZB = 48

BN = 1000
BE = 4000

_mesh = plsc.VectorSubcoreMesh(core_axis_name="c", subcore_axis_name="s")


def _dot(a, b):
    return jax.lax.dot_general(a, b, (((1,), (0,)), ((), ())),
                               preferred_element_type=jnp.float32)


def _full(shape):
    return pl.BlockSpec(shape, lambda i: tuple(0 for _ in shape))


def _node_init_body(x_ref, w1, b1, w2, b2, wa, wb, bm, p_ref, r_ref):
    h = jnp.maximum(_dot(x_ref[...], w1[...]) + b1[...], 0.0)
    nf = jnp.maximum(_dot(h, w2[...]) + b2[...], 0.0)
    p_ref[...] = _dot(nf, wa[...]) + bm[...]
    r_ref[...] = _dot(nf, wb[...])


def _node_init(x, w1, b1, w2, b2, wa, wb, bm):
    return pl.pallas_call(
        _node_init_body,
        grid=(N // BN,),
        in_specs=[pl.BlockSpec((BN, D), lambda i: (i, 0)),
                  _full((D, HID)), _full((1, HID)), _full((HID, D)), _full((1, D)),
                  _full((D, D)), _full((D, D)), _full((1, D))],
        out_specs=[pl.BlockSpec((BN, D), lambda i: (i, 0)),
                   pl.BlockSpec((BN, D), lambda i: (i, 0))],
        out_shape=[jax.ShapeDtypeStruct((N, D), jnp.float32),
                   jax.ShapeDtypeStruct((N, D), jnp.float32)],
    )(x, w1, b1, w2, b2, wa, wb, bm)


def _edge_init_body(ea_ref, w1, b1, w2, b2, wc, q_ref):
    h = jnp.maximum(_dot(ea_ref[...], w1[...]) + b1[...], 0.0)
    ef = jnp.maximum(_dot(h, w2[...]) + b2[...], 0.0)
    q_ref[...] = _dot(ef, wc[...])


def _edge_init(ea, w1, b1, w2, b2, wc):
    return pl.pallas_call(
        _edge_init_body,
        grid=(E // BE,),
        in_specs=[pl.BlockSpec((BE, 16), lambda i: (i, 0)),
                  _full((16, HID)), _full((1, HID)), _full((HID, D)), _full((1, D)),
                  _full((D, D))],
        out_specs=pl.BlockSpec((BE, D), lambda i: (i, 0)),
        out_shape=jax.ShapeDtypeStruct((E, D), jnp.float32),
    )(ea, w1, b1, w2, b2, wc)


def _edge_step_body(with_q, with_head, hs_ref, q_ref, w2, b2, wc,
                    wh1, bh1, wh2, bh2, *outs):
    h = jnp.maximum(hs_ref[...] + q_ref[...], 0.0)
    e = jnp.maximum(_dot(h, w2[...]) + b2[...], 0.0)
    outs[0][...] = e
    k = 1
    if with_q:
        outs[k][...] = _dot(e, wc[...])
        k += 1
    if with_head:
        hc = jnp.maximum(_dot(e, wh1[...]) + bh1[...], 0.0)
        outs[k][...] = _dot(hc, wh2[...]) + bh2[...]


def _edge_step(hs, q, w2, b2, wc, wh1, bh1, wh2, bh2, with_q, with_head):
    eb = lambda: pl.BlockSpec((BE, D), lambda i: (i, 0))
    out_specs = [eb()]
    out_shape = [jax.ShapeDtypeStruct((E, D), jnp.float32)]
    if with_q:
        out_specs.append(eb())
        out_shape.append(jax.ShapeDtypeStruct((E, D), jnp.float32))
    if with_head:
        out_specs.append(pl.BlockSpec((BE, 1), lambda i: (i, 0)))
        out_shape.append(jax.ShapeDtypeStruct((E, 1), jnp.float32))
    return pl.pallas_call(
        functools.partial(_edge_step_body, with_q, with_head),
        grid=(E // BE,),
        in_specs=[eb(), eb(),
                  _full((HID, D)), _full((1, D)), _full((D, D)),
                  _full((D, HC)), _full((1, HC)), _full((HC, 1)), _full((1, 1))],
        out_specs=out_specs,
        out_shape=out_shape,
    )(hs, q, w2, b2, wc, wh1, bh1, wh2, bh2)


def _node_step_body(p0_ref, p1_ref, wa, wb, bm, p_ref, r_ref):
    nf = p0_ref[...] + p1_ref[...]
    p_ref[...] = _dot(nf, wa[...]) + bm[...]
    r_ref[...] = _dot(nf, wb[...])


def _node_step(p0, p1, wa, wb, bm):
    nb = lambda: pl.BlockSpec((BN, D), lambda i: (i, 0))
    return pl.pallas_call(
        _node_step_body,
        grid=(N // BN,),
        in_specs=[nb(), nb(), _full((D, D)), _full((D, D)), _full((1, D))],
        out_specs=[nb(), nb()],
        out_shape=[jax.ShapeDtypeStruct((N, D), jnp.float32),
                   jax.ShapeDtypeStruct((N, D), jnp.float32)],
    )(p0, p1, wa, wb, bm)


def _final_body(p0_ref, p1_ref, wn1, bn1, wn2, bn2, wc1, bc1, wc2, bc2,
                pn_ref, pc_ref):
    nf = p0_ref[...] + p1_ref[...]
    hn = jnp.maximum(_dot(nf, wn1[...]) + bn1[...], 0.0)
    pn_ref[...] = _dot(hn, wn2[...]) + bn2[...]
    hc = jnp.maximum(_dot(nf, wc1[...]) + bc1[...], 0.0)
    pc_ref[...] = _dot(hc, wc2[...]) + bc2[...]


def _final(p0, p1, wn1, bn1, wn2, bn2, wc1, bc1, wc2, bc2):
    nb = lambda: pl.BlockSpec((BN, D), lambda i: (i, 0))
    return pl.pallas_call(
        _final_body,
        grid=(N // BN,),
        in_specs=[nb(), nb(),
                  _full((D, HC)), _full((1, HC)), _full((HC, 1)), _full((1, 1)),
                  _full((D, HC)), _full((1, HC)), _full((HC, NCLS)), _full((1, NCLS))],
        out_specs=[pl.BlockSpec((BN, 1), lambda i: (i, 0)),
                   pl.BlockSpec((BN, NCLS), lambda i: (i, 0))],
        out_shape=[jax.ShapeDtypeStruct((N, 1), jnp.float32),
                   jax.ShapeDtypeStruct((N, NCLS), jnp.float32)],
    )(p0, p1, wn1, bn1, wn2, bn2, wc1, bc1, wc2, bc2)


GBUF = 5
GBUF_S = 2


@functools.partial(
    pl.kernel, mesh=_mesh,
    out_type=jax.ShapeDtypeStruct((E, D), jnp.float32),
    scratch_types=[pltpu.VMEM((NCH, C), jnp.int32),
                   pltpu.VMEM((NCH, C), jnp.int32),
                   pltpu.VMEM((GBUF, C, D), jnp.float32),
                   pltpu.VMEM((GBUF, C, D), jnp.float32),
                   pltpu.SemaphoreType.DMA,
                   pltpu.SemaphoreType.DMA],
)
def _sc_gather(p_hbm, r_hbm, src_hbm, dst_hbm, hs_hbm,
               si_v, di_v, pr_v, rr_v, gsem, wsem):
    wid = lax.axis_index("s") * 2 + lax.axis_index("c")
    base = wid * EPW
    pltpu.sync_copy(src_hbm.at[wid], si_v)
    pltpu.sync_copy(dst_hbm.at[wid], di_v)

    def group(g, carry):
        cps = []
        for b in range(GBUF):
            j = g * GBUF + b
            ofs = base + j * C

            @pl.when(g > 0)
            def _():
                pltpu.make_async_copy(pr_v.at[b], hs_hbm.at[pl.ds(ofs, C)],
                                      wsem).wait()

            cps.append((pltpu.async_copy(p_hbm.at[si_v.at[j]], pr_v.at[b], gsem),
                        pltpu.async_copy(r_hbm.at[di_v.at[j]], rr_v.at[b], gsem)))
        for b in range(GBUF):
            j = g * GBUF + b
            ofs = base + j * C
            cps[b][0].wait()
            cps[b][1].wait()

            def vsum(t, carry2):
                i = t // (D // 16)
                k = (t % (D // 16)) * 16
                pr_v[b, i, pl.ds(k, 16)] = (pr_v[b, i, pl.ds(k, 16)]
                                            + rr_v[b, i, pl.ds(k, 16)])
                return carry2

            lax.fori_loop(0, C * (D // 16), vsum, 0, unroll=False)
            pltpu.async_copy(pr_v.at[b], hs_hbm.at[pl.ds(ofs, C)], wsem)
        return carry

    lax.fori_loop(0, NCH // GBUF, group, 0, unroll=False)
    for b in range(GBUF):
        pltpu.make_async_copy(pr_v.at[b], hs_hbm.at[pl.ds(base, C)], wsem).wait()


@functools.partial(
    pl.kernel, mesh=_mesh,
    out_type=jax.ShapeDtypeStruct((2, N, D), jnp.float32),
    scratch_types=[pltpu.VMEM((NCH, C), jnp.int32),
                   pltpu.VMEM((GBUF_S, C, D), jnp.float32),
                   pltpu.VMEM((ZB, D), jnp.float32),
                   pltpu.VMEM_SHARED((N, D), jnp.float32),
                   pltpu.SemaphoreType.DMA,
                   pltpu.SemaphoreType.DMA],
)
def _sc_scatter(e_hbm, dst_hbm, out_hbm, di_v, rows_v, z_v, acc_sh, rsem, asem):
    cid = lax.axis_index("c")
    sid = lax.axis_index("s")
    wid = sid * 2 + cid
    base = wid * EPW

    zvec = jnp.zeros((16,), jnp.float32)

    def zfill(t, carry):
        i = t // (D // 16)
        k = t % (D // 16)
        z_v[i, pl.ds(k * 16, 16)] = zvec
        return carry

    lax.fori_loop(0, ZB * (D // 16), zfill, 0, unroll=False)
    for m in range(STRIPE // ZB):
        pltpu.sync_copy(z_v, acc_sh.at[pl.ds(sid * STRIPE + m * ZB, ZB)])

    @pl.when(sid == TPS - 1)
    def _():
        pltpu.sync_copy(z_v.at[pl.ds(0, 16)], acc_sh.at[pl.ds(TPS * STRIPE, 16)])

    plsc.subcore_barrier()

    pltpu.sync_copy(dst_hbm.at[wid], di_v)

    def group(g, carry):
        cps = []
        for b in range(GBUF_S):
            j = g * GBUF_S + b
            ofs = base + j * C

            @pl.when(g > 0)
            def _():
                pltpu.make_async_copy(rows_v.at[b], acc_sh.at[di_v.at[j]],
                                      asem).wait()

            cps.append(pltpu.async_copy(e_hbm.at[pl.ds(ofs, C)],
                                        rows_v.at[b], rsem))
        for b in range(GBUF_S):
            j = g * GBUF_S + b
            cps[b].wait()
            pltpu.async_copy(rows_v.at[b], acc_sh.at[di_v.at[j]], asem,
                             add=True)
        return carry

    lax.fori_loop(0, NCH // GBUF_S, group, 0, unroll=False)
    for b in range(GBUF_S):
        pltpu.make_async_copy(rows_v.at[b], acc_sh.at[di_v.at[b]], asem).wait()
    plsc.subcore_barrier()

    for m in range(STRIPE // ZB):
        pltpu.sync_copy(acc_sh.at[pl.ds(sid * STRIPE + m * ZB, ZB)],
                        out_hbm.at[cid, pl.ds(sid * STRIPE + m * ZB, ZB)])

    @pl.when(sid == TPS - 1)
    def _():
        pltpu.sync_copy(acc_sh.at[pl.ds(TPS * STRIPE, 16)],
                        out_hbm.at[cid, pl.ds(TPS * STRIPE, 16)])


def kernel(x, edge_attr, edge_index, node_types,
           Wne1, bne1, Wne2, bne2,
           Wee1, bee1, Wee2, bee2,
           Wm1, bm1, Wm2, bm2,
           Wec1, bec1, Wec2, bec2,
           Wnc1, bnc1, Wnc2, bnc2,
           Wc1, bc1, Wc2, bc2):
    src3 = edge_index[0].reshape(NW, NCH, C)
    dst3 = edge_index[1].reshape(NW, NCH, C)
    wa, wb, wc = Wm1[:D], Wm1[D:2 * D], Wm1[2 * D:]
    row = lambda v: v.reshape(1, -1)

    p, r = _node_init(x, Wne1, row(bne1), Wne2, row(bne2), wa, wb, row(bm1))
    q = _edge_init(edge_attr, Wee1, row(bee1), Wee2, row(bee2), wc)

    pe = None
    part = None
    for t in range(3):
        hs = _sc_gather(p, r, src3, dst3)
        outs = _edge_step(hs, q, Wm2, row(bm2), wc,
                          Wec1, row(bec1), Wec2, row(bec2),
                          with_q=(t < 2), with_head=(t == 1))
        e_new = outs[0]
        if t < 2:
            q = outs[1]
        if t == 1:
            pe = outs[2]
        part = _sc_scatter(e_new, dst3)
        if t < 2:
            p, r = _node_step(part[0], part[1], wa, wb, row(bm1))

    pn, pc = _final(part[0], part[1], Wnc1, row(bnc1), Wnc2, row(bnc2),
                    Wc1, row(bc1), Wc2, row(bc2))
    return (pe.reshape(E), pn.reshape(N), pc)


# speedup vs baseline: 1.1954x; 1.1954x over previous
import functools

import jax
import jax.numpy as jnp
from jax import lax
from jax.experimental import pallas as pl
from jax.experimental.pallas import tpu as pltpu
from jax.experimental.pallas import tpu_sc as plsc

N = 10000
E = 320000
D = 128
HID = 128
HC = 64
NCLS = 8

NW = 32
EPW = E // NW
C = 40
NCH = EPW // C
TPS = 16
STRIPE = 624
ZB = 48

BN = 1000
BE = 4000

_mesh = plsc.VectorSubcoreMesh(core_axis_name="c", subcore_axis_name="s")


def _dot(a, b):
    return jax.lax.dot_general(a, b, (((1,), (0,)), ((), ())),
                               preferred_element_type=jnp.float32)


def _full(shape):
    return pl.BlockSpec(shape, lambda i: tuple(0 for _ in shape))


def _node_init_body(x_ref, w1, b1, w2, b2, wa, wb, bm, p_ref, r_ref):
    h = jnp.maximum(_dot(x_ref[...], w1[...]) + b1[...], 0.0)
    nf = jnp.maximum(_dot(h, w2[...]) + b2[...], 0.0)
    p_ref[...] = _dot(nf, wa[...]) + bm[...]
    r_ref[...] = _dot(nf, wb[...])


def _node_init(x, w1, b1, w2, b2, wa, wb, bm):
    return pl.pallas_call(
        _node_init_body,
        grid=(N // BN,),
        in_specs=[pl.BlockSpec((BN, D), lambda i: (i, 0)),
                  _full((D, HID)), _full((1, HID)), _full((HID, D)), _full((1, D)),
                  _full((D, D)), _full((D, D)), _full((1, D))],
        out_specs=[pl.BlockSpec((BN, D), lambda i: (i, 0)),
                   pl.BlockSpec((BN, D), lambda i: (i, 0))],
        out_shape=[jax.ShapeDtypeStruct((N, D), jnp.float32),
                   jax.ShapeDtypeStruct((N, D), jnp.float32)],
    )(x, w1, b1, w2, b2, wa, wb, bm)


def _edge_init_body(ea_ref, w1, b1, w2, b2, ef_ref):
    h = jnp.maximum(_dot(ea_ref[...], w1[...]) + b1[...], 0.0)
    ef_ref[...] = jnp.maximum(_dot(h, w2[...]) + b2[...], 0.0)


def _edge_init(ea, w1, b1, w2, b2):
    return pl.pallas_call(
        _edge_init_body,
        grid=(E // BE,),
        in_specs=[pl.BlockSpec((BE, 16), lambda i: (i, 0)),
                  _full((16, HID)), _full((1, HID)), _full((HID, D)), _full((1, D))],
        out_specs=pl.BlockSpec((BE, D), lambda i: (i, 0)),
        out_shape=jax.ShapeDtypeStruct((E, D), jnp.float32),
    )(ea, w1, b1, w2, b2)


def _edge_step_body(with_head, pg_ref, rg_ref, ep_ref, w2, b2, wc,
                    wh1, bh1, wh2, bh2, *outs):
    q = _dot(ep_ref[...], wc[...])
    h = jnp.maximum(pg_ref[...] + rg_ref[...] + q, 0.0)
    e = jnp.maximum(_dot(h, w2[...]) + b2[...], 0.0)
    outs[0][...] = e
    if with_head:
        hc = jnp.maximum(_dot(e, wh1[...]) + bh1[...], 0.0)
        outs[1][...] = _dot(hc, wh2[...]) + bh2[...]


def _edge_step(pg, rg, ep, w2, b2, wc, wh1, bh1, wh2, bh2, with_head):
    eb = lambda: pl.BlockSpec((BE, D), lambda i: (i, 0))
    out_specs = [eb()]
    out_shape = [jax.ShapeDtypeStruct((E, D), jnp.float32)]
    if with_head:
        out_specs.append(pl.BlockSpec((BE, 1), lambda i: (i, 0)))
        out_shape.append(jax.ShapeDtypeStruct((E, 1), jnp.float32))
    return pl.pallas_call(
        functools.partial(_edge_step_body, with_head),
        grid=(E // BE,),
        in_specs=[eb(), eb(), eb(),
                  _full((HID, D)), _full((1, D)), _full((D, D)),
                  _full((D, HC)), _full((1, HC)), _full((HC, 1)), _full((1, 1))],
        out_specs=out_specs,
        out_shape=out_shape,
    )(pg, rg, ep, w2, b2, wc, wh1, bh1, wh2, bh2)


def _node_step_body(p0_ref, p1_ref, wa, wb, bm, p_ref, r_ref):
    nf = p0_ref[...] + p1_ref[...]
    p_ref[...] = _dot(nf, wa[...]) + bm[...]
    r_ref[...] = _dot(nf, wb[...])


def _node_step(p0, p1, wa, wb, bm):
    nb = lambda: pl.BlockSpec((BN, D), lambda i: (i, 0))
    return pl.pallas_call(
        _node_step_body,
        grid=(N // BN,),
        in_specs=[nb(), nb(), _full((D, D)), _full((D, D)), _full((1, D))],
        out_specs=[nb(), nb()],
        out_shape=[jax.ShapeDtypeStruct((N, D), jnp.float32),
                   jax.ShapeDtypeStruct((N, D), jnp.float32)],
    )(p0, p1, wa, wb, bm)


def _final_body(p0_ref, p1_ref, wn1, bn1, wn2, bn2, wc1, bc1, wc2, bc2,
                pn_ref, pc_ref):
    nf = p0_ref[...] + p1_ref[...]
    hn = jnp.maximum(_dot(nf, wn1[...]) + bn1[...], 0.0)
    pn_ref[...] = _dot(hn, wn2[...]) + bn2[...]
    hc = jnp.maximum(_dot(nf, wc1[...]) + bc1[...], 0.0)
    pc_ref[...] = _dot(hc, wc2[...]) + bc2[...]


def _final(p0, p1, wn1, bn1, wn2, bn2, wc1, bc1, wc2, bc2):
    nb = lambda: pl.BlockSpec((BN, D), lambda i: (i, 0))
    return pl.pallas_call(
        _final_body,
        grid=(N // BN,),
        in_specs=[nb(), nb(),
                  _full((D, HC)), _full((1, HC)), _full((HC, 1)), _full((1, 1)),
                  _full((D, HC)), _full((1, HC)), _full((HC, NCLS)), _full((1, NCLS))],
        out_specs=[pl.BlockSpec((BN, 1), lambda i: (i, 0)),
                   pl.BlockSpec((BN, NCLS), lambda i: (i, 0))],
        out_shape=[jax.ShapeDtypeStruct((N, 1), jnp.float32),
                   jax.ShapeDtypeStruct((N, NCLS), jnp.float32)],
    )(p0, p1, wn1, bn1, wn2, bn2, wc1, bc1, wc2, bc2)


GBUF = 5
GBUF_S = 2


@functools.partial(
    pl.kernel, mesh=_mesh,
    out_type=[jax.ShapeDtypeStruct((E, D), jnp.float32),
              jax.ShapeDtypeStruct((E, D), jnp.float32)],
    scratch_types=[pltpu.VMEM((NCH, C), jnp.int32),
                   pltpu.VMEM((NCH, C), jnp.int32),
                   pltpu.VMEM((GBUF, C, D), jnp.float32),
                   pltpu.VMEM((GBUF, C, D), jnp.float32),
                   pltpu.SemaphoreType.DMA,
                   pltpu.SemaphoreType.DMA],
)
def _sc_gather(p_hbm, r_hbm, src_hbm, dst_hbm, pg_hbm, rg_hbm,
               si_v, di_v, pr_v, rr_v, gsem, wsem):
    wid = lax.axis_index("s") * 2 + lax.axis_index("c")
    base = wid * EPW
    pltpu.sync_copy(src_hbm.at[wid], si_v)
    pltpu.sync_copy(dst_hbm.at[wid], di_v)

    def group(g, carry):
        cps = []
        for b in range(GBUF):
            j = g * GBUF + b
            ofs = base + j * C

            @pl.when(g > 0)
            def _():
                pltpu.make_async_copy(pr_v.at[b], pg_hbm.at[pl.ds(ofs, C)],
                                      wsem).wait()
                pltpu.make_async_copy(rr_v.at[b], rg_hbm.at[pl.ds(ofs, C)],
                                      wsem).wait()

            cps.append((pltpu.async_copy(p_hbm.at[si_v.at[j]], pr_v.at[b], gsem),
                        pltpu.async_copy(r_hbm.at[di_v.at[j]], rr_v.at[b], gsem)))
        for b in range(GBUF):
            j = g * GBUF + b
            ofs = base + j * C
            cps[b][0].wait()
            cps[b][1].wait()
            pltpu.async_copy(pr_v.at[b], pg_hbm.at[pl.ds(ofs, C)], wsem)
            pltpu.async_copy(rr_v.at[b], rg_hbm.at[pl.ds(ofs, C)], wsem)
        return carry

    lax.fori_loop(0, NCH // GBUF, group, 0, unroll=False)
    for b in range(GBUF):
        pltpu.make_async_copy(pr_v.at[b], pg_hbm.at[pl.ds(base, C)], wsem).wait()
        pltpu.make_async_copy(rr_v.at[b], rg_hbm.at[pl.ds(base, C)], wsem).wait()


@functools.partial(
    pl.kernel, mesh=_mesh,
    out_type=jax.ShapeDtypeStruct((2, N, D), jnp.float32),
    scratch_types=[pltpu.VMEM((NCH, C), jnp.int32),
                   pltpu.VMEM((GBUF_S, C, D), jnp.float32),
                   pltpu.VMEM((ZB, D), jnp.float32),
                   pltpu.VMEM_SHARED((N, D), jnp.float32),
                   pltpu.SemaphoreType.DMA,
                   pltpu.SemaphoreType.DMA],
)
def _sc_scatter(e_hbm, dst_hbm, out_hbm, di_v, rows_v, z_v, acc_sh, rsem, asem):
    cid = lax.axis_index("c")
    sid = lax.axis_index("s")
    wid = sid * 2 + cid
    base = wid * EPW

    zvec = jnp.zeros((16,), jnp.float32)

    def zfill(t, carry):
        i = t // (D // 16)
        k = t % (D // 16)
        z_v[i, pl.ds(k * 16, 16)] = zvec
        return carry

    lax.fori_loop(0, ZB * (D // 16), zfill, 0, unroll=False)
    for m in range(STRIPE // ZB):
        pltpu.sync_copy(z_v, acc_sh.at[pl.ds(sid * STRIPE + m * ZB, ZB)])

    @pl.when(sid == TPS - 1)
    def _():
        pltpu.sync_copy(z_v.at[pl.ds(0, 16)], acc_sh.at[pl.ds(TPS * STRIPE, 16)])

    plsc.subcore_barrier()

    pltpu.sync_copy(dst_hbm.at[wid], di_v)

    def group(g, carry):
        cps = []
        for b in range(GBUF_S):
            j = g * GBUF_S + b
            ofs = base + j * C

            @pl.when(g > 0)
            def _():
                pltpu.make_async_copy(rows_v.at[b], acc_sh.at[di_v.at[j]],
                                      asem).wait()

            cps.append(pltpu.async_copy(e_hbm.at[pl.ds(ofs, C)],
                                        rows_v.at[b], rsem))
        for b in range(GBUF_S):
            j = g * GBUF_S + b
            cps[b].wait()
            pltpu.async_copy(rows_v.at[b], acc_sh.at[di_v.at[j]], asem,
                             add=True)
        return carry

    lax.fori_loop(0, NCH // GBUF_S, group, 0, unroll=False)
    for b in range(GBUF_S):
        pltpu.make_async_copy(rows_v.at[b], acc_sh.at[di_v.at[b]], asem).wait()
    plsc.subcore_barrier()

    for m in range(STRIPE // ZB):
        pltpu.sync_copy(acc_sh.at[pl.ds(sid * STRIPE + m * ZB, ZB)],
                        out_hbm.at[cid, pl.ds(sid * STRIPE + m * ZB, ZB)])

    @pl.when(sid == TPS - 1)
    def _():
        pltpu.sync_copy(acc_sh.at[pl.ds(TPS * STRIPE, 16)],
                        out_hbm.at[cid, pl.ds(TPS * STRIPE, 16)])


def kernel(x, edge_attr, edge_index, node_types,
           Wne1, bne1, Wne2, bne2,
           Wee1, bee1, Wee2, bee2,
           Wm1, bm1, Wm2, bm2,
           Wec1, bec1, Wec2, bec2,
           Wnc1, bnc1, Wnc2, bnc2,
           Wc1, bc1, Wc2, bc2):
    src3 = edge_index[0].reshape(NW, NCH, C)
    dst3 = edge_index[1].reshape(NW, NCH, C)
    wa, wb, wc = Wm1[:D], Wm1[D:2 * D], Wm1[2 * D:]
    row = lambda v: v.reshape(1, -1)

    p, r = _node_init(x, Wne1, row(bne1), Wne2, row(bne2), wa, wb, row(bm1))
    e_prev = _edge_init(edge_attr, Wee1, row(bee1), Wee2, row(bee2))

    pe = None
    part = None
    for t in range(3):
        pg, rg = _sc_gather(p, r, src3, dst3)
        outs = _edge_step(pg, rg, e_prev, Wm2, row(bm2), wc,
                          Wec1, row(bec1), Wec2, row(bec2),
                          with_head=(t == 1))
        e_prev = outs[0]
        if t == 1:
            pe = outs[1]
        part = _sc_scatter(e_prev, dst3)
        if t < 2:
            p, r = _node_step(part[0], part[1], wa, wb, row(bm1))

    pn, pc = _final(part[0], part[1], Wnc1, row(bnc1), Wnc2, row(bnc2),
                    Wc1, row(bc1), Wc2, row(bc2))
    return (pe.reshape(E), pn.reshape(N), pc)


# speedup vs baseline: 1.2616x; 1.0554x over previous
import functools

import jax
import jax.numpy as jnp
from jax import lax
from jax.experimental import pallas as pl
from jax.experimental.pallas import tpu as pltpu
from jax.experimental.pallas import tpu_sc as plsc

N = 10000
E = 320000
D = 128
HID = 128
HC = 64
NCLS = 8
DE = 16

NW = 32
EPW = E // NW
C = 40
NCH = EPW // C
TPS = 16
STRIPE = 624
ZB = 48

BN = 2000
BE = 4000

_mesh = plsc.VectorSubcoreMesh(core_axis_name="c", subcore_axis_name="s")


def _dot(a, b):
    return jax.lax.dot_general(a, b, (((1,), (0,)), ((), ())),
                               preferred_element_type=jnp.float32)


def _full(shape):
    return pl.BlockSpec(shape, lambda i: tuple(0 for _ in shape))


def _node_init_body(x_ref, w1, b1, w2, b2, wa, wb, bm, p_ref, r_ref):
    h = jnp.maximum(_dot(x_ref[...], w1[...]) + b1[...], 0.0)
    nf = jnp.maximum(_dot(h, w2[...]) + b2[...], 0.0)
    p_ref[...] = _dot(nf, wa[...]) + bm[...]
    r_ref[...] = _dot(nf, wb[...])


def _node_init(x, w1, b1, w2, b2, wa, wb, bm):
    return pl.pallas_call(
        _node_init_body,
        grid=(N // BN,),
        in_specs=[pl.BlockSpec((BN, D), lambda i: (i, 0)),
                  _full((D, HID)), _full((1, HID)), _full((HID, D)), _full((1, D)),
                  _full((D, D)), _full((D, D)), _full((1, D))],
        out_specs=[pl.BlockSpec((BN, D), lambda i: (i, 0)),
                   pl.BlockSpec((BN, D), lambda i: (i, 0))],
        out_shape=[jax.ShapeDtypeStruct((N, D), jnp.float32),
                   jax.ShapeDtypeStruct((N, D), jnp.float32)],
    )(x, w1, b1, w2, b2, wa, wb, bm)


def _edge_step0_body(pg_ref, rg_ref, ea_ref, we1, be1, we2, be2, wc,
                     w2, b2, e_ref):
    t = jnp.maximum(_dot(ea_ref[...], we1[...]) + be1[...], 0.0)
    ef = jnp.maximum(_dot(t, we2[...]) + be2[...], 0.0)
    q = _dot(ef, wc[...])
    h = jnp.maximum(pg_ref[...] + rg_ref[...] + q, 0.0)
    e_ref[...] = jnp.maximum(_dot(h, w2[...]) + b2[...], 0.0)


def _edge_step0(pg, rg, ea, we1, be1, we2, be2, wc, w2, b2):
    eb = lambda: pl.BlockSpec((BE, D), lambda i: (i, 0))
    return pl.pallas_call(
        _edge_step0_body,
        grid=(E // BE,),
        in_specs=[eb(), eb(), pl.BlockSpec((BE, DE), lambda i: (i, 0)),
                  _full((DE, HID)), _full((1, HID)), _full((HID, D)), _full((1, D)),
                  _full((D, D)), _full((HID, D)), _full((1, D))],
        out_specs=eb(),
        out_shape=jax.ShapeDtypeStruct((E, D), jnp.float32),
    )(pg, rg, ea, we1, be1, we2, be2, wc, w2, b2)


def _edge_step_body(with_head, pg_ref, rg_ref, ep_ref, w2, b2, wc,
                    wh1, bh1, wh2, bh2, *outs):
    q = _dot(ep_ref[...], wc[...])
    h = jnp.maximum(pg_ref[...] + rg_ref[...] + q, 0.0)
    e = jnp.maximum(_dot(h, w2[...]) + b2[...], 0.0)
    outs[0][...] = e
    if with_head:
        hc = jnp.maximum(_dot(e, wh1[...]) + bh1[...], 0.0)
        outs[1][...] = _dot(hc, wh2[...]) + bh2[...]


def _edge_step(pg, rg, ep, w2, b2, wc, wh1, bh1, wh2, bh2, with_head):
    eb = lambda: pl.BlockSpec((BE, D), lambda i: (i, 0))
    out_specs = [eb()]
    out_shape = [jax.ShapeDtypeStruct((E, D), jnp.float32)]
    if with_head:
        out_specs.append(pl.BlockSpec((BE, 1), lambda i: (i, 0)))
        out_shape.append(jax.ShapeDtypeStruct((E, 1), jnp.float32))
    return pl.pallas_call(
        functools.partial(_edge_step_body, with_head),
        grid=(E // BE,),
        in_specs=[eb(), eb(), eb(),
                  _full((HID, D)), _full((1, D)), _full((D, D)),
                  _full((D, HC)), _full((1, HC)), _full((HC, 1)), _full((1, 1))],
        out_specs=out_specs,
        out_shape=out_shape,
    )(pg, rg, ep, w2, b2, wc, wh1, bh1, wh2, bh2)


def _node_step_body(p0_ref, p1_ref, wa, wb, bm, p_ref, r_ref):
    nf = p0_ref[...] + p1_ref[...]
    p_ref[...] = _dot(nf, wa[...]) + bm[...]
    r_ref[...] = _dot(nf, wb[...])


def _node_step(p0, p1, wa, wb, bm):
    nb = lambda: pl.BlockSpec((BN, D), lambda i: (i, 0))
    return pl.pallas_call(
        _node_step_body,
        grid=(N // BN,),
        in_specs=[nb(), nb(), _full((D, D)), _full((D, D)), _full((1, D))],
        out_specs=[nb(), nb()],
        out_shape=[jax.ShapeDtypeStruct((N, D), jnp.float32),
                   jax.ShapeDtypeStruct((N, D), jnp.float32)],
    )(p0, p1, wa, wb, bm)


def _final_body(p0_ref, p1_ref, wn1, bn1, wn2, bn2, wc1, bc1, wc2, bc2,
                pn_ref, pc_ref):
    nf = p0_ref[...] + p1_ref[...]
    hn = jnp.maximum(_dot(nf, wn1[...]) + bn1[...], 0.0)
    pn_ref[...] = _dot(hn, wn2[...]) + bn2[...]
    hc = jnp.maximum(_dot(nf, wc1[...]) + bc1[...], 0.0)
    pc_ref[...] = _dot(hc, wc2[...]) + bc2[...]


def _final(p0, p1, wn1, bn1, wn2, bn2, wc1, bc1, wc2, bc2):
    nb = lambda: pl.BlockSpec((BN, D), lambda i: (i, 0))
    return pl.pallas_call(
        _final_body,
        grid=(N // BN,),
        in_specs=[nb(), nb(),
                  _full((D, HC)), _full((1, HC)), _full((HC, 1)), _full((1, 1)),
                  _full((D, HC)), _full((1, HC)), _full((HC, NCLS)), _full((1, NCLS))],
        out_specs=[pl.BlockSpec((BN, 1), lambda i: (i, 0)),
                   pl.BlockSpec((BN, NCLS), lambda i: (i, 0))],
        out_shape=[jax.ShapeDtypeStruct((N, 1), jnp.float32),
                   jax.ShapeDtypeStruct((N, NCLS), jnp.float32)],
    )(p0, p1, wn1, bn1, wn2, bn2, wc1, bc1, wc2, bc2)


GBUF = 5
GBUF_S = 2


@functools.partial(
    pl.kernel, mesh=_mesh,
    out_type=[jax.ShapeDtypeStruct((E, D), jnp.float32),
              jax.ShapeDtypeStruct((E, D), jnp.float32)],
    scratch_types=[pltpu.VMEM((NCH, C), jnp.int32),
                   pltpu.VMEM((NCH, C), jnp.int32),
                   pltpu.VMEM((GBUF, C, D), jnp.float32),
                   pltpu.VMEM((GBUF, C, D), jnp.float32),
                   pltpu.SemaphoreType.DMA,
                   pltpu.SemaphoreType.DMA],
)
def _sc_gather(p_hbm, r_hbm, src_hbm, dst_hbm, pg_hbm, rg_hbm,
               si_v, di_v, pr_v, rr_v, gsem, wsem):
    wid = lax.axis_index("s") * 2 + lax.axis_index("c")
    base = wid * EPW
    pltpu.sync_copy(src_hbm.at[wid], si_v)
    pltpu.sync_copy(dst_hbm.at[wid], di_v)

    def group(g, carry):
        cps = []
        for b in range(GBUF):
            j = g * GBUF + b
            ofs = base + j * C

            @pl.when(g > 0)
            def _():
                pltpu.make_async_copy(pr_v.at[b], pg_hbm.at[pl.ds(ofs, C)],
                                      wsem).wait()
                pltpu.make_async_copy(rr_v.at[b], rg_hbm.at[pl.ds(ofs, C)],
                                      wsem).wait()

            cps.append((pltpu.async_copy(p_hbm.at[si_v.at[j]], pr_v.at[b], gsem),
                        pltpu.async_copy(r_hbm.at[di_v.at[j]], rr_v.at[b], gsem)))
        for b in range(GBUF):
            j = g * GBUF + b
            ofs = base + j * C
            cps[b][0].wait()
            cps[b][1].wait()
            pltpu.async_copy(pr_v.at[b], pg_hbm.at[pl.ds(ofs, C)], wsem)
            pltpu.async_copy(rr_v.at[b], rg_hbm.at[pl.ds(ofs, C)], wsem)
        return carry

    lax.fori_loop(0, NCH // GBUF, group, 0, unroll=False)
    for b in range(GBUF):
        pltpu.make_async_copy(pr_v.at[b], pg_hbm.at[pl.ds(base, C)], wsem).wait()
        pltpu.make_async_copy(rr_v.at[b], rg_hbm.at[pl.ds(base, C)], wsem).wait()


@functools.partial(
    pl.kernel, mesh=_mesh,
    out_type=jax.ShapeDtypeStruct((2, N, D), jnp.float32),
    scratch_types=[pltpu.VMEM((NCH, C), jnp.int32),
                   pltpu.VMEM((GBUF_S, C, D), jnp.float32),
                   pltpu.VMEM((ZB, D), jnp.float32),
                   pltpu.VMEM_SHARED((N, D), jnp.float32),
                   pltpu.SemaphoreType.DMA,
                   pltpu.SemaphoreType.DMA],
)
def _sc_scatter(e_hbm, dst_hbm, out_hbm, di_v, rows_v, z_v, acc_sh, rsem, asem):
    cid = lax.axis_index("c")
    sid = lax.axis_index("s")
    wid = sid * 2 + cid
    base = wid * EPW

    zvec = jnp.zeros((16,), jnp.float32)

    def zfill(t, carry):
        i = t // (D // 16)
        k = t % (D // 16)
        z_v[i, pl.ds(k * 16, 16)] = zvec
        return carry

    lax.fori_loop(0, ZB * (D // 16), zfill, 0, unroll=False)
    for m in range(STRIPE // ZB):
        pltpu.sync_copy(z_v, acc_sh.at[pl.ds(sid * STRIPE + m * ZB, ZB)])

    @pl.when(sid == TPS - 1)
    def _():
        pltpu.sync_copy(z_v.at[pl.ds(0, 16)], acc_sh.at[pl.ds(TPS * STRIPE, 16)])

    plsc.subcore_barrier()

    pltpu.sync_copy(dst_hbm.at[wid], di_v)

    def group(g, carry):
        cps = []
        for b in range(GBUF_S):
            j = g * GBUF_S + b
            ofs = base + j * C

            @pl.when(g > 0)
            def _():
                pltpu.make_async_copy(rows_v.at[b], acc_sh.at[di_v.at[j]],
                                      asem).wait()

            cps.append(pltpu.async_copy(e_hbm.at[pl.ds(ofs, C)],
                                        rows_v.at[b], rsem))
        for b in range(GBUF_S):
            j = g * GBUF_S + b
            cps[b].wait()
            pltpu.async_copy(rows_v.at[b], acc_sh.at[di_v.at[j]], asem,
                             add=True)
        return carry

    lax.fori_loop(0, NCH // GBUF_S, group, 0, unroll=False)
    for b in range(GBUF_S):
        pltpu.make_async_copy(rows_v.at[b], acc_sh.at[di_v.at[b]], asem).wait()
    plsc.subcore_barrier()

    for m in range(STRIPE // ZB):
        pltpu.sync_copy(acc_sh.at[pl.ds(sid * STRIPE + m * ZB, ZB)],
                        out_hbm.at[cid, pl.ds(sid * STRIPE + m * ZB, ZB)])

    @pl.when(sid == TPS - 1)
    def _():
        pltpu.sync_copy(acc_sh.at[pl.ds(TPS * STRIPE, 16)],
                        out_hbm.at[cid, pl.ds(TPS * STRIPE, 16)])


def kernel(x, edge_attr, edge_index, node_types,
           Wne1, bne1, Wne2, bne2,
           Wee1, bee1, Wee2, bee2,
           Wm1, bm1, Wm2, bm2,
           Wec1, bec1, Wec2, bec2,
           Wnc1, bnc1, Wnc2, bnc2,
           Wc1, bc1, Wc2, bc2):
    src3 = edge_index[0].reshape(NW, NCH, C)
    dst3 = edge_index[1].reshape(NW, NCH, C)
    wa, wb, wc = Wm1[:D], Wm1[D:2 * D], Wm1[2 * D:]
    row = lambda v: v.reshape(1, -1)

    p, r = _node_init(x, Wne1, row(bne1), Wne2, row(bne2), wa, wb, row(bm1))

    pe = None
    part = None
    e_prev = None
    for t in range(3):
        pg, rg = _sc_gather(p, r, src3, dst3)
        if t == 0:
            e_prev = _edge_step0(pg, rg, edge_attr, Wee1, row(bee1),
                                 Wee2, row(bee2), wc, Wm2, row(bm2))
        else:
            outs = _edge_step(pg, rg, e_prev, Wm2, row(bm2), wc,
                              Wec1, row(bec1), Wec2, row(bec2),
                              with_head=(t == 1))
            e_prev = outs[0]
            if t == 1:
                pe = outs[1]
        part = _sc_scatter(e_prev, dst3)
        if t < 2:
            p, r = _node_step(part[0], part[1], wa, wb, row(bm1))

    pn, pc = _final(part[0], part[1], Wnc1, row(bnc1), Wnc2, row(bnc2),
                    Wc1, row(bc1), Wc2, row(bc2))
    return (pe.reshape(E), pn.reshape(N), pc)
